# Initial kernel scaffold; baseline (speedup 1.0000x reference)
import jax
import jax.numpy as jnp
from jax.experimental import pallas as pl


def kernel(x, edge_index, edge_attr, W1, b1, W2, b2, eps):
    raise NotImplementedError("write your pallas kernel here")


# speedup vs baseline: 5.2336x; 5.2336x over previous
import functools

import jax
import jax.numpy as jnp
from jax import lax
from jax.experimental import pallas as pl
from jax.experimental.pallas import tpu as pltpu
from jax.experimental.pallas import tpu_sc as plsc

N_NODES = 10000
N_EDGES = 320000
D_FEAT = 128

NUM_CORES = 2
NUM_SUBCORES = 16
NUM_WORKERS = NUM_CORES * NUM_SUBCORES

EDGES_PER_WORKER = N_EDGES // NUM_WORKERS
CHUNK = 80
NUM_CHUNKS = EDGES_PER_WORKER // CHUNK
ROWS_PER_TILE = 624
ROWS_TAIL = N_NODES - ROWS_PER_TILE * NUM_SUBCORES


def _sc_aggregate(x, src, dst, zeros):
    mesh = plsc.VectorSubcoreMesh(
        core_axis_name="c", subcore_axis_name="s",
        num_cores=NUM_CORES, num_subcores=NUM_SUBCORES)

    @functools.partial(
        pl.kernel,
        mesh=mesh,
        out_type=jax.ShapeDtypeStruct((NUM_CORES, N_NODES, D_FEAT), jnp.float32),
        scratch_types=[
            pltpu.VMEM((CHUNK,), jnp.int32),
            pltpu.VMEM((CHUNK,), jnp.int32),
            pltpu.VMEM((CHUNK, D_FEAT), jnp.float32),
            pltpu.VMEM_SHARED((N_NODES, D_FEAT), jnp.float32),
            pltpu.SemaphoreType.DMA,
        ],
    )
    def k(x_hbm, src_hbm, dst_hbm, zeros_hbm, out_hbm,
          src_v, dst_v, rows_v, acc, sem):
        c = lax.axis_index("c")
        s = lax.axis_index("s")
        wid = c * NUM_SUBCORES + s

        row0 = s * ROWS_PER_TILE
        tail0 = ROWS_PER_TILE * NUM_SUBCORES
        pltpu.sync_copy(zeros_hbm.at[pl.ds(row0, ROWS_PER_TILE)],
                        acc.at[pl.ds(row0, ROWS_PER_TILE)])

        @pl.when(s == 0)
        def _():
            pltpu.sync_copy(zeros_hbm.at[pl.ds(tail0, ROWS_TAIL)],
                            acc.at[pl.ds(tail0, ROWS_TAIL)])

        plsc.subcore_barrier()

        base = wid * EDGES_PER_WORKER

        def body(i, _):
            off = base + i * CHUNK
            pltpu.sync_copy(src_hbm.at[pl.ds(off, CHUNK)], src_v)
            pltpu.sync_copy(dst_hbm.at[pl.ds(off, CHUNK)], dst_v)
            pltpu.async_copy(x_hbm.at[src_v], rows_v, sem).wait()
            pltpu.sync_copy(rows_v, acc.at[dst_v], add=True)
            return ()

        lax.fori_loop(0, NUM_CHUNKS, body, ())

        plsc.subcore_barrier()
        pltpu.sync_copy(acc.at[pl.ds(row0, ROWS_PER_TILE)],
                        out_hbm.at[c].at[pl.ds(row0, ROWS_PER_TILE)])

        @pl.when(s == 0)
        def _():
            pltpu.sync_copy(acc.at[pl.ds(tail0, ROWS_TAIL)],
                            out_hbm.at[c].at[pl.ds(tail0, ROWS_TAIL)])

    return k(x, src, dst, zeros)


def _mlp_body(x_ref, a0_ref, a1_ref, w1_ref, b1_ref, w2_ref, b2_ref,
              scale_ref, out_ref):
    h = x_ref[...] * scale_ref[0] + a0_ref[...] + a1_ref[...]
    h = jnp.dot(h, w1_ref[...], preferred_element_type=jnp.float32)
    h = jnp.maximum(h + b1_ref[...], 0.0)
    o = jnp.dot(h, w2_ref[...], preferred_element_type=jnp.float32)
    out_ref[...] = o + b2_ref[...]


def _tc_mlp(x, a0, a1, W1, b1, W2, b2, scale):
    BR = 400
    grid = (N_NODES // BR,)
    row_spec = pl.BlockSpec((BR, D_FEAT), lambda i: (i, 0))
    full_spec = pl.BlockSpec((D_FEAT, D_FEAT), lambda i: (0, 0))
    bias_spec = pl.BlockSpec((1, D_FEAT), lambda i: (0, 0))
    scale_spec = pl.BlockSpec(memory_space=pltpu.SMEM)
    return pl.pallas_call(
        _mlp_body,
        grid=grid,
        in_specs=[row_spec, row_spec, row_spec, full_spec, bias_spec,
                  full_spec, bias_spec, scale_spec],
        out_specs=row_spec,
        out_shape=jax.ShapeDtypeStruct((N_NODES, D_FEAT), jnp.float32),
    )(x, a0, a1, W1, b1.reshape(1, -1), W2, b2.reshape(1, -1), scale)


def kernel(x, edge_index, edge_attr, W1, b1, W2, b2, eps):
    src = edge_index[0]
    dst = edge_index[1]
    zeros = jnp.zeros((N_NODES, D_FEAT), dtype=jnp.float32)
    aggs = _sc_aggregate(x, src, dst, zeros)
    scale = (1.0 + eps).reshape(1).astype(jnp.float32)
    return _tc_mlp(x, aggs[0], aggs[1], W1, b1, W2, b2, scale)


# speedup vs baseline: 10.0984x; 1.9295x over previous
import functools

import jax
import jax.numpy as jnp
from jax import lax
from jax.experimental import pallas as pl
from jax.experimental.pallas import tpu as pltpu
from jax.experimental.pallas import tpu_sc as plsc

N_NODES = 10000
N_EDGES = 320000
D_FEAT = 128

NUM_CORES = 2
NUM_SUBCORES = 16
NUM_WORKERS = NUM_CORES * NUM_SUBCORES

EDGES_PER_WORKER = N_EDGES // NUM_WORKERS
CHUNK = 40
NUM_CHUNKS = EDGES_PER_WORKER // CHUNK
ROWS_PER_TILE = 624
ROWS_TAIL = N_NODES - ROWS_PER_TILE * NUM_SUBCORES


NBUF = 5
IDX_AHEAD = 4
GATHER_AHEAD = 3


def _sc_aggregate(x, idx4d, zeros):
    mesh = plsc.VectorSubcoreMesh(
        core_axis_name="c", subcore_axis_name="s",
        num_cores=NUM_CORES, num_subcores=NUM_SUBCORES)

    rows_t = pltpu.VMEM((CHUNK, D_FEAT), jnp.float32)
    idx_t = pltpu.VMEM((2, CHUNK), jnp.int32)

    @functools.partial(
        pl.kernel,
        mesh=mesh,
        out_type=jax.ShapeDtypeStruct((NUM_CORES, N_NODES, D_FEAT), jnp.float32),
        scratch_types=[
            [idx_t] * NBUF,
            [rows_t] * NBUF,
            pltpu.VMEM_SHARED((N_NODES, D_FEAT), jnp.float32),
            [pltpu.SemaphoreType.DMA] * NBUF,
            [pltpu.SemaphoreType.DMA] * NBUF,
        ],
    )
    def k(x_hbm, idx_hbm, zeros_hbm, out_hbm,
          idxb, rows, acc, isems, gsems):
        c = lax.axis_index("c")
        s = lax.axis_index("s")
        wid = c * NUM_SUBCORES + s

        row0 = s * ROWS_PER_TILE
        tail0 = ROWS_PER_TILE * NUM_SUBCORES
        pltpu.sync_copy(zeros_hbm.at[pl.ds(row0, ROWS_PER_TILE)],
                        acc.at[pl.ds(row0, ROWS_PER_TILE)])

        @pl.when(s == 0)
        def _():
            pltpu.sync_copy(zeros_hbm.at[pl.ds(tail0, ROWS_TAIL)],
                            acc.at[pl.ds(tail0, ROWS_TAIL)])

        plsc.subcore_barrier()

        def fire_idx(chunk, b):
            pltpu.async_copy(idx_hbm.at[wid].at[chunk], idxb[b], isems[b])

        def wait_idx(b):
            pltpu.make_async_copy(idx_hbm.at[wid].at[0], idxb[b],
                                  isems[b]).wait()

        def fire_gather(b):
            pltpu.async_copy(x_hbm.at[idxb[b].at[0]], rows[b], gsems[b])

        def wait_gather(b):
            pltpu.make_async_copy(x_hbm.at[idxb[b].at[0]], rows[b],
                                  gsems[b]).wait()

        for b in range(IDX_AHEAD):
            fire_idx(b, b)
        for b in range(GATHER_AHEAD):
            wait_idx(b)
            fire_gather(b)

        def body(g, _):
            for b in range(NBUF):
                chunk = g * NBUF + b

                @pl.when(chunk + IDX_AHEAD < NUM_CHUNKS)
                def _():
                    fire_idx(chunk + IDX_AHEAD, (b + IDX_AHEAD) % NBUF)

                @pl.when(chunk + GATHER_AHEAD < NUM_CHUNKS)
                def _():
                    bg = (b + GATHER_AHEAD) % NBUF
                    wait_idx(bg)
                    fire_gather(bg)

                wait_gather(b)
                pltpu.sync_copy(rows[b], acc.at[idxb[b].at[1]], add=True)
            return ()

        lax.fori_loop(0, NUM_CHUNKS // NBUF, body, ())

        plsc.subcore_barrier()
        pltpu.sync_copy(acc.at[pl.ds(row0, ROWS_PER_TILE)],
                        out_hbm.at[c].at[pl.ds(row0, ROWS_PER_TILE)])

        @pl.when(s == 0)
        def _():
            pltpu.sync_copy(acc.at[pl.ds(tail0, ROWS_TAIL)],
                            out_hbm.at[c].at[pl.ds(tail0, ROWS_TAIL)])

    return k(x, idx4d, zeros)


def _mlp_body(x_ref, a0_ref, a1_ref, w1_ref, b1_ref, w2_ref, b2_ref,
              scale_ref, out_ref):
    h = x_ref[...] * scale_ref[0] + a0_ref[...] + a1_ref[...]
    h = jnp.dot(h, w1_ref[...], preferred_element_type=jnp.float32)
    h = jnp.maximum(h + b1_ref[...], 0.0)
    o = jnp.dot(h, w2_ref[...], preferred_element_type=jnp.float32)
    out_ref[...] = o + b2_ref[...]


def _tc_mlp(x, a0, a1, W1, b1, W2, b2, scale):
    BR = 400
    grid = (N_NODES // BR,)
    row_spec = pl.BlockSpec((BR, D_FEAT), lambda i: (i, 0))
    full_spec = pl.BlockSpec((D_FEAT, D_FEAT), lambda i: (0, 0))
    bias_spec = pl.BlockSpec((1, D_FEAT), lambda i: (0, 0))
    scale_spec = pl.BlockSpec(memory_space=pltpu.SMEM)
    return pl.pallas_call(
        _mlp_body,
        grid=grid,
        in_specs=[row_spec, row_spec, row_spec, full_spec, bias_spec,
                  full_spec, bias_spec, scale_spec],
        out_specs=row_spec,
        out_shape=jax.ShapeDtypeStruct((N_NODES, D_FEAT), jnp.float32),
    )(x, a0, a1, W1, b1.reshape(1, -1), W2, b2.reshape(1, -1), scale)


def kernel(x, edge_index, edge_attr, W1, b1, W2, b2, eps):
    idx4d = (edge_index
             .reshape(2, NUM_WORKERS, NUM_CHUNKS, CHUNK)
             .transpose(1, 2, 0, 3))
    zeros = jnp.zeros((N_NODES, D_FEAT), dtype=jnp.float32)
    aggs = _sc_aggregate(x, idx4d, zeros)
    scale = (1.0 + eps).reshape(1).astype(jnp.float32)
    return _tc_mlp(x, aggs[0], aggs[1], W1, b1, W2, b2, scale)


# speedup vs baseline: 11.6509x; 1.1537x over previous
import functools

import jax
import jax.numpy as jnp
import numpy as np
from jax import lax
from jax.experimental import pallas as pl
from jax.experimental.pallas import tpu as pltpu
from jax.experimental.pallas import tpu_sc as plsc

N_NODES = 10000
N_EDGES = 320000
D_FEAT = 128

NUM_CORES = 2
NUM_SUBCORES = 16
NUM_WORKERS = NUM_CORES * NUM_SUBCORES

EDGES_PER_WORKER = N_EDGES // NUM_WORKERS
CHUNK = 40
NUM_CHUNKS = EDGES_PER_WORKER // CHUNK
ROWS_PER_TILE = 624
ROWS_TAIL = N_NODES - ROWS_PER_TILE * NUM_SUBCORES

NBUF = 5
IDX_AHEAD = 4
GATHER_AHEAD = 3

_ZEROS = np.zeros((N_NODES, D_FEAT), dtype=np.float32)


def _sc_aggregate(x, src, dst, zeros):
    mesh = plsc.VectorSubcoreMesh(
        core_axis_name="c", subcore_axis_name="s",
        num_cores=NUM_CORES, num_subcores=NUM_SUBCORES)

    rows_t = pltpu.VMEM((CHUNK, D_FEAT), jnp.float32)
    idx_t = pltpu.VMEM((CHUNK,), jnp.int32)
    part_t = jax.ShapeDtypeStruct((N_NODES, D_FEAT), jnp.float32)

    @functools.partial(
        pl.kernel,
        mesh=mesh,
        out_type=(part_t, part_t),
        scratch_types=[
            [idx_t] * NBUF,
            [idx_t] * NBUF,
            [rows_t] * NBUF,
            pltpu.VMEM_SHARED((N_NODES, D_FEAT), jnp.float32),
            [pltpu.SemaphoreType.DMA] * NBUF,
            [pltpu.SemaphoreType.DMA] * NBUF,
            [pltpu.SemaphoreType.DMA] * NBUF,
        ],
    )
    def k(x_hbm, src_hbm, dst_hbm, zeros_hbm, out0_hbm, out1_hbm,
          srcb, dstb, rows, acc, isems, gsems, ssems):
        c = lax.axis_index("c")
        s = lax.axis_index("s")
        wid = c * NUM_SUBCORES + s

        row0 = s * ROWS_PER_TILE
        tail0 = ROWS_PER_TILE * NUM_SUBCORES
        pltpu.sync_copy(zeros_hbm.at[pl.ds(row0, ROWS_PER_TILE)],
                        acc.at[pl.ds(row0, ROWS_PER_TILE)])

        @pl.when(s == 0)
        def _():
            pltpu.sync_copy(zeros_hbm.at[pl.ds(tail0, ROWS_TAIL)],
                            acc.at[pl.ds(tail0, ROWS_TAIL)])

        plsc.subcore_barrier()

        base = wid * EDGES_PER_WORKER

        def fire_idx(chunk, b):
            off = base + chunk * CHUNK
            pltpu.async_copy(src_hbm.at[pl.ds(off, CHUNK)], srcb[b], isems[b])
            pltpu.async_copy(dst_hbm.at[pl.ds(off, CHUNK)], dstb[b], isems[b])

        def wait_idx(b):
            pltpu.make_async_copy(src_hbm.at[pl.ds(0, CHUNK)], srcb[b],
                                  isems[b]).wait()
            pltpu.make_async_copy(dst_hbm.at[pl.ds(0, CHUNK)], dstb[b],
                                  isems[b]).wait()

        def fire_gather(b):
            pltpu.async_copy(x_hbm.at[srcb[b]], rows[b], gsems[b])

        def wait_gather(b):
            pltpu.make_async_copy(x_hbm.at[srcb[b]], rows[b], gsems[b]).wait()

        def fire_scatter(b):
            pltpu.async_copy(rows[b], acc.at[dstb[b]], ssems[b], add=True)

        def wait_scatter(b):
            pltpu.make_async_copy(rows[b], acc.at[dstb[b]], ssems[b]).wait()

        for b in range(IDX_AHEAD):
            fire_idx(b, b)
        for b in range(GATHER_AHEAD):
            wait_idx(b)
            fire_gather(b)

        def body(g, _):
            for b in range(NBUF):
                chunk = g * NBUF + b

                @pl.when(chunk >= 1)
                def _():
                    wait_scatter((b - 1) % NBUF)

                @pl.when(chunk + IDX_AHEAD < NUM_CHUNKS)
                def _():
                    fire_idx(chunk + IDX_AHEAD, (b + IDX_AHEAD) % NBUF)

                @pl.when(chunk + GATHER_AHEAD < NUM_CHUNKS)
                def _():
                    bg = (b + GATHER_AHEAD) % NBUF
                    wait_idx(bg)
                    fire_gather(bg)

                wait_gather(b)
                fire_scatter(b)
            return ()

        lax.fori_loop(0, NUM_CHUNKS // NBUF, body, ())
        wait_scatter((NUM_CHUNKS - 1) % NBUF)

        plsc.subcore_barrier()

        @pl.when(c == 0)
        def _():
            pltpu.sync_copy(acc.at[pl.ds(row0, ROWS_PER_TILE)],
                            out0_hbm.at[pl.ds(row0, ROWS_PER_TILE)])

            @pl.when(s == 0)
            def _():
                pltpu.sync_copy(acc.at[pl.ds(tail0, ROWS_TAIL)],
                                out0_hbm.at[pl.ds(tail0, ROWS_TAIL)])

        @pl.when(c == 1)
        def _():
            pltpu.sync_copy(acc.at[pl.ds(row0, ROWS_PER_TILE)],
                            out1_hbm.at[pl.ds(row0, ROWS_PER_TILE)])

            @pl.when(s == 0)
            def _():
                pltpu.sync_copy(acc.at[pl.ds(tail0, ROWS_TAIL)],
                                out1_hbm.at[pl.ds(tail0, ROWS_TAIL)])

    return k(x, src, dst, zeros)


def _mlp_body(x_ref, a0_ref, a1_ref, w1_ref, b1_ref, w2_ref, b2_ref,
              scale_ref, out_ref):
    h = x_ref[...] * scale_ref[0] + a0_ref[...] + a1_ref[...]
    h = jnp.dot(h, w1_ref[...], preferred_element_type=jnp.float32)
    h = jnp.maximum(h + b1_ref[...], 0.0)
    o = jnp.dot(h, w2_ref[...], preferred_element_type=jnp.float32)
    out_ref[...] = o + b2_ref[...]


def _tc_mlp(x, a0, a1, W1, b1, W2, b2, scale):
    BR = 400
    grid = (N_NODES // BR,)
    row_spec = pl.BlockSpec((BR, D_FEAT), lambda i: (i, 0))
    full_spec = pl.BlockSpec((D_FEAT, D_FEAT), lambda i: (0, 0))
    bias_spec = pl.BlockSpec((1, D_FEAT), lambda i: (0, 0))
    scale_spec = pl.BlockSpec(memory_space=pltpu.SMEM)
    return pl.pallas_call(
        _mlp_body,
        grid=grid,
        in_specs=[row_spec, row_spec, row_spec, full_spec, bias_spec,
                  full_spec, bias_spec, scale_spec],
        out_specs=row_spec,
        out_shape=jax.ShapeDtypeStruct((N_NODES, D_FEAT), jnp.float32),
    )(x, a0, a1, W1, b1.reshape(1, -1), W2, b2.reshape(1, -1), scale)


def kernel(x, edge_index, edge_attr, W1, b1, W2, b2, eps):
    src = edge_index[0]
    dst = edge_index[1]
    zeros = jnp.asarray(_ZEROS)
    a0, a1 = _sc_aggregate(x, src, dst, zeros)
    scale = (1.0 + eps).reshape(1).astype(jnp.float32)
    return _tc_mlp(x, a0, a1, W1, b1, W2, b2, scale)


# speedup vs baseline: 14.1796x; 1.2170x over previous
import functools

import jax
import jax.numpy as jnp
import numpy as np
from jax import lax
from jax.experimental import pallas as pl
from jax.experimental.pallas import tpu as pltpu
from jax.experimental.pallas import tpu_sc as plsc

N_NODES = 10000
N_EDGES = 320000
D_FEAT = 128

NUM_CORES = 2
NUM_SUBCORES = 16
NUM_WORKERS = NUM_CORES * NUM_SUBCORES

EDGES_PER_WORKER = N_EDGES // NUM_WORKERS
CHUNK = 40
NUM_CHUNKS = EDGES_PER_WORKER // CHUNK
ROWS_PER_TILE = 624
ROWS_TAIL = N_NODES - ROWS_PER_TILE * NUM_SUBCORES

NBUF = 5
NDST = 2 * NBUF
IDX_AHEAD = 4
GATHER_AHEAD = 3
SCAT_DRAIN = 2

_ZEROS = np.zeros((N_NODES, D_FEAT), dtype=np.float32)


def _sc_aggregate(x, edges, zeros):
    mesh = plsc.VectorSubcoreMesh(
        core_axis_name="c", subcore_axis_name="s",
        num_cores=NUM_CORES, num_subcores=NUM_SUBCORES)

    rows_t = pltpu.VMEM((CHUNK, D_FEAT), jnp.float32)
    idx_t = pltpu.VMEM((CHUNK,), jnp.int32)
    part_t = jax.ShapeDtypeStruct((N_NODES, D_FEAT), jnp.float32)

    @functools.partial(
        pl.kernel,
        mesh=mesh,
        out_type=(part_t, part_t),
        scratch_types=[
            [idx_t] * NBUF,
            [idx_t] * NDST,
            [rows_t] * NBUF,
            pltpu.VMEM_SHARED((N_NODES, D_FEAT), jnp.float32),
            [pltpu.SemaphoreType.DMA] * NBUF,
            [pltpu.SemaphoreType.DMA] * NBUF,
            [pltpu.SemaphoreType.DMA] * NBUF,
        ],
    )
    def k(x_hbm, edge_hbm, zeros_hbm, out0_hbm, out1_hbm,
          srcb, dstb, rows, acc, isems, gsems, ssems):
        c = lax.axis_index("c")
        s = lax.axis_index("s")
        wid = c * NUM_SUBCORES + s

        base = wid * EDGES_PER_WORKER

        def fire_idx(chunk, bs, bd):
            off = base + chunk * CHUNK
            pltpu.async_copy(edge_hbm.at[pl.ds(off, CHUNK)],
                             srcb[bs], isems[bs])
            pltpu.async_copy(edge_hbm.at[pl.ds(N_EDGES + off, CHUNK)],
                             dstb[bd], isems[bs])

        def wait_idx(bs, bd):
            pltpu.make_async_copy(edge_hbm.at[pl.ds(0, CHUNK)], srcb[bs],
                                  isems[bs]).wait()
            pltpu.make_async_copy(edge_hbm.at[pl.ds(0, CHUNK)], dstb[bd],
                                  isems[bs]).wait()

        def fire_gather(b):
            pltpu.async_copy(x_hbm.at[srcb[b]], rows[b], gsems[b])

        def wait_gather(b):
            pltpu.make_async_copy(x_hbm.at[srcb[b]], rows[b], gsems[b]).wait()

        def fire_scatter(b, bd):
            pltpu.async_copy(rows[b], acc.at[dstb[bd]], ssems[b], add=True)

        def wait_scatter(b, bd):
            pltpu.make_async_copy(rows[b], acc.at[dstb[bd]], ssems[b]).wait()

        for b in range(IDX_AHEAD):
            fire_idx(b, b, b)

        row0 = s * ROWS_PER_TILE
        tail0 = ROWS_PER_TILE * NUM_SUBCORES
        pltpu.sync_copy(zeros_hbm.at[pl.ds(row0, ROWS_PER_TILE)],
                        acc.at[pl.ds(row0, ROWS_PER_TILE)])

        @pl.when(s == 0)
        def _():
            pltpu.sync_copy(zeros_hbm.at[pl.ds(tail0, ROWS_TAIL)],
                            acc.at[pl.ds(tail0, ROWS_TAIL)])

        for b in range(GATHER_AHEAD):
            wait_idx(b, b)
            fire_gather(b)

        plsc.subcore_barrier()

        def body(g, _):
            for b in range(NDST):
                chunk = g * NDST + b

                @pl.when(chunk >= SCAT_DRAIN)
                def _():
                    wait_scatter((b - SCAT_DRAIN) % NBUF,
                                 (b - SCAT_DRAIN) % NDST)

                @pl.when(chunk + IDX_AHEAD < NUM_CHUNKS)
                def _():
                    fire_idx(chunk + IDX_AHEAD, (b + IDX_AHEAD) % NBUF,
                             (b + IDX_AHEAD) % NDST)

                @pl.when(chunk + GATHER_AHEAD < NUM_CHUNKS)
                def _():
                    bg = (b + GATHER_AHEAD) % NBUF
                    wait_idx(bg, (b + GATHER_AHEAD) % NDST)
                    fire_gather(bg)

                wait_gather(b % NBUF)
                fire_scatter(b % NBUF, b % NDST)
            return ()

        lax.fori_loop(0, NUM_CHUNKS // NDST, body, ())
        for back in range(SCAT_DRAIN, 0, -1):
            wait_scatter((NUM_CHUNKS - back) % NBUF, (NUM_CHUNKS - back) % NDST)

        plsc.subcore_barrier()

        @pl.when(c == 0)
        def _():
            pltpu.sync_copy(acc.at[pl.ds(row0, ROWS_PER_TILE)],
                            out0_hbm.at[pl.ds(row0, ROWS_PER_TILE)])

            @pl.when(s == 0)
            def _():
                pltpu.sync_copy(acc.at[pl.ds(tail0, ROWS_TAIL)],
                                out0_hbm.at[pl.ds(tail0, ROWS_TAIL)])

        @pl.when(c == 1)
        def _():
            pltpu.sync_copy(acc.at[pl.ds(row0, ROWS_PER_TILE)],
                            out1_hbm.at[pl.ds(row0, ROWS_PER_TILE)])

            @pl.when(s == 0)
            def _():
                pltpu.sync_copy(acc.at[pl.ds(tail0, ROWS_TAIL)],
                                out1_hbm.at[pl.ds(tail0, ROWS_TAIL)])

    return k(x, edges, zeros)


def _mlp_body(x_ref, a0_ref, a1_ref, w1_ref, b1_ref, w2_ref, b2_ref,
              scale_ref, out_ref):
    h = x_ref[...] * scale_ref[0] + a0_ref[...] + a1_ref[...]
    h = jnp.dot(h, w1_ref[...], preferred_element_type=jnp.float32)
    h = jnp.maximum(h + b1_ref[...], 0.0)
    o = jnp.dot(h, w2_ref[...], preferred_element_type=jnp.float32)
    out_ref[...] = o + b2_ref[...]


def _tc_mlp(x, a0, a1, W1, b1, W2, b2, scale):
    BR = 1000
    grid = (N_NODES // BR,)
    row_spec = pl.BlockSpec((BR, D_FEAT), lambda i: (i, 0))
    full_spec = pl.BlockSpec((D_FEAT, D_FEAT), lambda i: (0, 0))
    bias_spec = pl.BlockSpec((1, D_FEAT), lambda i: (0, 0))
    scale_spec = pl.BlockSpec(memory_space=pltpu.SMEM)
    return pl.pallas_call(
        _mlp_body,
        grid=grid,
        in_specs=[row_spec, row_spec, row_spec, full_spec, bias_spec,
                  full_spec, bias_spec, scale_spec],
        out_specs=row_spec,
        out_shape=jax.ShapeDtypeStruct((N_NODES, D_FEAT), jnp.float32),
    )(x, a0, a1, W1, b1.reshape(1, -1), W2, b2.reshape(1, -1), scale)


def kernel(x, edge_index, edge_attr, W1, b1, W2, b2, eps):
    edges = edge_index.reshape(-1)
    zeros = jnp.asarray(_ZEROS)
    a0, a1 = _sc_aggregate(x, edges, zeros)
    scale = (1.0 + eps).reshape(1).astype(jnp.float32)
    return _tc_mlp(x, a0, a1, W1, b1, W2, b2, scale)
